# baseline (speedup 1.0000x reference)
import functools

import jax
import jax.numpy as jnp
from jax import lax
from jax.experimental import pallas as pl
from jax.experimental.pallas import tpu as pltpu
from jax.experimental.pallas import tpu_sc as plsc

D = 128
G = 128
_CH = 128
_NC = 2
_NS = 16
_NW = _NC * _NS
_P = 1


def _sc_segment_sum(z, src_a, dst_a, src_b, dst_b, zeros, n_acc):
    na = src_a.shape[1]
    nb = src_b.shape[1]
    nmax = max(na, nb)
    rows_per_tile = n_acc // _NS
    mesh = plsc.VectorSubcoreMesh(core_axis_name="c", subcore_axis_name="s")

    @functools.partial(
        pl.kernel,
        mesh=mesh,
        out_type=jax.ShapeDtypeStruct((_NC, n_acc, D), jnp.float32),
        scratch_types=[
            pltpu.VMEM((nmax, _CH), jnp.int32),
            pltpu.VMEM((nmax, _CH), jnp.int32),
            pltpu.VMEM((_CH, D), jnp.float32),
            pltpu.VMEM_SHARED((n_acc, D), jnp.float32),
            pltpu.SemaphoreType.DMA,
            pltpu.SemaphoreType.DMA,
        ],
    )
    def k(z_hbm, srca_hbm, dsta_hbm, srcb_hbm, dstb_hbm, zero_hbm, out_hbm,
          src_v, dst_v, buf, acc, gsem, ssem):
        c = lax.axis_index("c")
        s = lax.axis_index("s")
        r0 = s * rows_per_tile
        pltpu.sync_copy(zero_hbm.at[pl.ds(r0, rows_per_tile)],
                        acc.at[pl.ds(r0, rows_per_tile)])
        plsc.subcore_barrier()

        def g_start(j):
            pltpu.async_copy(z_hbm.at[src_v.at[j]], buf, gsem)

        def g_wait(j):
            pltpu.make_async_copy(z_hbm.at[src_v.at[j]], buf, gsem).wait()

        def s_start(j):
            pltpu.async_copy(buf, acc.at[dst_v.at[j]], ssem, add=True)

        def s_wait(j):
            pltpu.make_async_copy(buf, acc.at[dst_v.at[j]], ssem).wait()

        def run(src_h, dst_h, nck):
            pltpu.sync_copy(src_h.at[s], src_v.at[pl.ds(0, nck)])
            pltpu.sync_copy(dst_h.at[s], dst_v.at[pl.ds(0, nck)])
            g_start(0)

            def body(j, carry):
                g_wait(j)
                s_start(j)
                s_wait(j)
                g_start(j + 1)
                return carry

            lax.fori_loop(0, nck - 1, body, 0)
            g_wait(nck - 1)
            s_start(nck - 1)
            s_wait(nck - 1)

        @pl.when(c == 0)
        def _():
            run(srca_hbm, dsta_hbm, na)

        @pl.when(c == 1)
        def _():
            run(srcb_hbm, dstb_hbm, nb)

        plsc.subcore_barrier()
        pltpu.sync_copy(acc.at[pl.ds(r0, rows_per_tile)],
                        out_hbm.at[c, pl.ds(r0, rows_per_tile)])

    return k(z, src_a, dst_a, src_b, dst_b, zeros)


def _tc_layer(z, agg, batch2d, w1, b1, w2, b2, gamma, beta, n):
    def body(z_ref, agg_ref, batch_ref, w1_ref, b1_ref, w2_ref, b2_ref,
             gm_ref, bt_ref, z_out, g_out):
        h = z_ref[...] + agg_ref[0, :n, :] + agg_ref[1, :n, :]
        h = jnp.maximum(
            jnp.dot(h, w1_ref[...], preferred_element_type=jnp.float32)
            + b1_ref[...], 0.0)
        h = jnp.dot(h, w2_ref[...], preferred_element_type=jnp.float32) + b2_ref[...]
        h = jnp.maximum(h, 0.0)
        mean = jnp.mean(h, axis=0, keepdims=True)
        hc = h - mean
        var = jnp.mean(hc * hc, axis=0, keepdims=True)
        hn = hc * lax.rsqrt(var + 1e-5) * gm_ref[...] + bt_ref[...]
        z_out[...] = hn
        onehot = (batch_ref[...] == lax.broadcasted_iota(jnp.int32, (n, G), 1)
                  ).astype(jnp.float32)
        g_out[...] = lax.dot_general(onehot, hn, (((0,), (0,)), ((), ())),
                                     preferred_element_type=jnp.float32)

    return pl.pallas_call(
        body,
        out_shape=[
            jax.ShapeDtypeStruct((n, D), jnp.float32),
            jax.ShapeDtypeStruct((G, D), jnp.float32),
        ],
    )(z, agg, batch2d, w1, b1, w2, b2, gamma, beta)


def kernel(x, edge_index, batch,
           w1_0, b1_0, w2_0, b2_0, gamma_0, beta_0,
           w1_1, b1_1, w2_1, b2_1, gamma_1, beta_1,
           w1_2, b1_2, w2_2, b2_2, gamma_2, beta_2):
    n = x.shape[0]
    e = edge_index.shape[1]
    frac = 0.60
    total_chunks = -(-e // (_NS * _CH))
    na = max(1, min(total_chunks - 1, round(frac * e / (_NS * _CH)) - 4))
    nb = total_chunks - na + (1 if total_chunks * _NS * _CH < e else 0)
    while _NS * _CH * (na + nb) < e:
        nb += 1
    ea = _NS * _CH * na
    e_pad = _NS * _CH * (na + nb)
    n_acc = -(-(n + 1) // (_NS * 8)) * (_NS * 8)
    pad = e_pad - e
    src_all = jnp.concatenate([edge_index[0], jnp.zeros((pad,), jnp.int32)])
    dst_all = jnp.concatenate(
        [edge_index[1], n + jnp.arange(pad, dtype=jnp.int32) % (n_acc - n)])
    src_a = src_all[:ea].reshape(_NS, na, _CH)
    dst_a = dst_all[:ea].reshape(_NS, na, _CH)
    src_b = src_all[ea:].reshape(_NS, nb, _CH)
    dst_b = dst_all[ea:].reshape(_NS, nb, _CH)
    zeros = jnp.zeros((n_acc, D), jnp.float32)
    batch2d = batch.reshape(n, 1)

    params = [
        (w1_0, b1_0, w2_0, b2_0, gamma_0, beta_0),
        (w1_1, b1_1, w2_1, b2_1, gamma_1, beta_1),
        (w1_2, b1_2, w2_2, b2_2, gamma_2, beta_2),
    ]
    z = x
    zs, gs = [], []
    for (w1, b1, w2, b2, gamma, beta) in params:
        agg = _sc_segment_sum(z, src_a, dst_a, src_b, dst_b, zeros, n_acc)
        z, gp = _tc_layer(z, agg, batch2d, w1, b1.reshape(1, D), w2,
                          b2.reshape(1, D), gamma.reshape(1, D),
                          beta.reshape(1, D), n)
        zs.append(z)
        gs.append(gp)
    return jnp.concatenate(zs, axis=1), jnp.concatenate(gs, axis=1)


# speedup vs baseline: 1.0168x; 1.0168x over previous
import functools

import jax
import jax.numpy as jnp
from jax import lax
from jax.experimental import pallas as pl
from jax.experimental.pallas import tpu as pltpu
from jax.experimental.pallas import tpu_sc as plsc

D = 128
G = 128
_CH = 128
_NC = 2
_NS = 16
_NW = _NC * _NS
_P = 1


def _sc_segment_sum(z, src_a, dst_a, src_b, dst_b, zeros, n_acc):
    na = src_a.shape[1]
    nb = src_b.shape[1]
    nmax = max(na, nb)
    rows_per_tile = n_acc // _NS
    mesh = plsc.VectorSubcoreMesh(core_axis_name="c", subcore_axis_name="s")

    @functools.partial(
        pl.kernel,
        mesh=mesh,
        out_type=jax.ShapeDtypeStruct((_NC, n_acc, D), jnp.float32),
        scratch_types=[
            pltpu.VMEM((nmax, _CH), jnp.int32),
            pltpu.VMEM((nmax, _CH), jnp.int32),
            pltpu.VMEM((_CH, D), jnp.float32),
            pltpu.VMEM_SHARED((n_acc, D), jnp.float32),
            pltpu.SemaphoreType.DMA,
            pltpu.SemaphoreType.DMA,
        ],
    )
    def k(z_hbm, srca_hbm, dsta_hbm, srcb_hbm, dstb_hbm, zero_hbm, out_hbm,
          src_v, dst_v, buf, acc, gsem, ssem):
        c = lax.axis_index("c")
        s = lax.axis_index("s")
        r0 = s * rows_per_tile
        pltpu.sync_copy(zero_hbm.at[pl.ds(r0, rows_per_tile)],
                        acc.at[pl.ds(r0, rows_per_tile)])
        plsc.subcore_barrier()

        def g_start(j):
            pltpu.async_copy(z_hbm.at[src_v.at[j]], buf, gsem)

        def g_wait(j):
            pltpu.make_async_copy(z_hbm.at[src_v.at[j]], buf, gsem).wait()

        def s_start(j):
            pltpu.async_copy(buf, acc.at[dst_v.at[j]], ssem, add=True)

        def s_wait(j):
            pltpu.make_async_copy(buf, acc.at[dst_v.at[j]], ssem).wait()

        def run(src_h, dst_h, nck):
            pltpu.sync_copy(src_h.at[s], src_v.at[pl.ds(0, nck)])
            pltpu.sync_copy(dst_h.at[s], dst_v.at[pl.ds(0, nck)])
            g_start(0)

            def body(j, carry):
                g_wait(j)
                s_start(j)
                s_wait(j)
                g_start(j + 1)
                return carry

            lax.fori_loop(0, nck - 1, body, 0)
            g_wait(nck - 1)
            s_start(nck - 1)
            s_wait(nck - 1)

        @pl.when(c == 0)
        def _():
            run(srca_hbm, dsta_hbm, na)

        @pl.when(c == 1)
        def _():
            run(srcb_hbm, dstb_hbm, nb)

        plsc.subcore_barrier()
        pltpu.sync_copy(acc.at[pl.ds(r0, rows_per_tile)],
                        out_hbm.at[c, pl.ds(r0, rows_per_tile)])

    return k(z, src_a, dst_a, src_b, dst_b, zeros)


def _tc_layer(z, agg, batch2d, w1, b1, w2, b2, gamma, beta, n):
    def body(z_ref, agg_ref, batch_ref, w1_ref, b1_ref, w2_ref, b2_ref,
             gm_ref, bt_ref, z_out, g_out):
        h = z_ref[...] + agg_ref[0, :n, :] + agg_ref[1, :n, :]
        h = jnp.maximum(
            jnp.dot(h, w1_ref[...], preferred_element_type=jnp.float32)
            + b1_ref[...], 0.0)
        h = jnp.dot(h, w2_ref[...], preferred_element_type=jnp.float32) + b2_ref[...]
        h = jnp.maximum(h, 0.0)
        mean = jnp.mean(h, axis=0, keepdims=True)
        hc = h - mean
        var = jnp.mean(hc * hc, axis=0, keepdims=True)
        hn = hc * lax.rsqrt(var + 1e-5) * gm_ref[...] + bt_ref[...]
        z_out[...] = hn
        onehot = (batch_ref[...] == lax.broadcasted_iota(jnp.int32, (n, G), 1)
                  ).astype(jnp.float32)
        g_out[...] = lax.dot_general(onehot, hn, (((0,), (0,)), ((), ())),
                                     preferred_element_type=jnp.float32)

    return pl.pallas_call(
        body,
        out_shape=[
            jax.ShapeDtypeStruct((n, D), jnp.float32),
            jax.ShapeDtypeStruct((G, D), jnp.float32),
        ],
    )(z, agg, batch2d, w1, b1, w2, b2, gamma, beta)


def kernel(x, edge_index, batch,
           w1_0, b1_0, w2_0, b2_0, gamma_0, beta_0,
           w1_1, b1_1, w2_1, b2_1, gamma_1, beta_1,
           w1_2, b1_2, w2_2, b2_2, gamma_2, beta_2):
    n = x.shape[0]
    e = edge_index.shape[1]
    frac = 0.60
    total_chunks = -(-e // (_NS * _CH))
    na = max(1, min(total_chunks - 1, round(frac * e / (_NS * _CH)) - 2))
    nb = total_chunks - na + (1 if total_chunks * _NS * _CH < e else 0)
    while _NS * _CH * (na + nb) < e:
        nb += 1
    ea = _NS * _CH * na
    e_pad = _NS * _CH * (na + nb)
    n_acc = -(-(n + 1) // (_NS * 8)) * (_NS * 8)
    pad = e_pad - e
    src_all = jnp.concatenate([edge_index[0], jnp.zeros((pad,), jnp.int32)])
    dst_all = jnp.concatenate(
        [edge_index[1], n + jnp.arange(pad, dtype=jnp.int32) % (n_acc - n)])
    src_a = src_all[:ea].reshape(_NS, na, _CH)
    dst_a = dst_all[:ea].reshape(_NS, na, _CH)
    src_b = src_all[ea:].reshape(_NS, nb, _CH)
    dst_b = dst_all[ea:].reshape(_NS, nb, _CH)
    zeros = jnp.zeros((n_acc, D), jnp.float32)
    batch2d = batch.reshape(n, 1)

    params = [
        (w1_0, b1_0, w2_0, b2_0, gamma_0, beta_0),
        (w1_1, b1_1, w2_1, b2_1, gamma_1, beta_1),
        (w1_2, b1_2, w2_2, b2_2, gamma_2, beta_2),
    ]
    z = x
    zs, gs = [], []
    for (w1, b1, w2, b2, gamma, beta) in params:
        agg = _sc_segment_sum(z, src_a, dst_a, src_b, dst_b, zeros, n_acc)
        z, gp = _tc_layer(z, agg, batch2d, w1, b1.reshape(1, D), w2,
                          b2.reshape(1, D), gamma.reshape(1, D),
                          beta.reshape(1, D), n)
        zs.append(z)
        gs.append(gp)
    return jnp.concatenate(zs, axis=1), jnp.concatenate(gs, axis=1)
